# Initial kernel scaffold; baseline (speedup 1.0000x reference)
import jax
import jax.numpy as jnp
from jax.experimental import pallas as pl


def kernel(x, edge_index, edge_attr, u, batch, We1, be1, We2, be2, Wn1, bn1w, Wn2, bn2w, Wg1, bg1, Wg2, bg2, Wfc1, bfc1, gamma, beta, Wfc2, bfc2):
    raise NotImplementedError("write your pallas kernel here")


# speedup vs baseline: 6.6206x; 6.6206x over previous
import functools

import jax
import jax.numpy as jnp
from jax import lax
from jax.experimental import pallas as pl
from jax.experimental.pallas import tpu as pltpu
from jax.experimental.pallas import tpu_sc as plsc

_N = 10000
_NP = 10240
_E = 160000
_B = 8
_D1 = 128
_D2 = 64
_D3 = 640

_NC = 2
_NS = 16
_NW = _NC * _NS
_CH = 128
_NCHUNK = _E // _CH
_TRIPS = _NCHUNK // _NW + 1


def _sc_mesh():
    return plsc.VectorSubcoreMesh(core_axis_name="c", subcore_axis_name="s",
                                  num_cores=_NC, num_subcores=_NS)


def _nodeproj(x, wcat):
    bn = 2000

    def body(x_ref, w_ref, bc_ref):
        bc_ref[...] = jnp.dot(x_ref[...], w_ref[...],
                              preferred_element_type=jnp.float32)

    return pl.pallas_call(
        body,
        grid=(_N // bn,),
        in_specs=[pl.BlockSpec((bn, _D1), lambda i: (i, 0)),
                  pl.BlockSpec((_D1, _D1), lambda i: (0, 0))],
        out_specs=pl.BlockSpec((bn, _D1), lambda i: (i, 0)),
        out_shape=jax.ShapeDtypeStruct((_N, _D1), jnp.float32),
    )(x, wcat)


def _sc_gather(a_tab, bc_tab, row, col):
    @functools.partial(
        pl.kernel,
        out_type=[jax.ShapeDtypeStruct((_E, _D1), jnp.float32),
                  jax.ShapeDtypeStruct((_E, _D1), jnp.float32)],
        mesh=_sc_mesh(),
        scratch_types=[
            pltpu.VMEM((_CH,), jnp.int32),
            pltpu.VMEM((_CH,), jnp.int32),
            pltpu.VMEM((_CH, _D1), jnp.float32),
            pltpu.VMEM((_CH, _D1), jnp.float32),
            pltpu.SemaphoreType.DMA,
            pltpu.SemaphoreType.DMA,
        ],
    )
    def k(a_hbm, bc_hbm, row_hbm, col_hbm, g1_hbm, g2_hbm,
          idxr, idxc, r1, r2, sem1, sem2):
        wid = lax.axis_index("s") * _NC + lax.axis_index("c")

        @pl.loop(0, _TRIPS)
        def _trip(t):
            cidx = wid + t * _NW

            @pl.when(cidx < _NCHUNK)
            def _():
                off = cidx * _CH
                pltpu.sync_copy(row_hbm.at[pl.ds(off, _CH)], idxr)
                pltpu.sync_copy(col_hbm.at[pl.ds(off, _CH)], idxc)
                d1 = pltpu.async_copy(a_hbm.at[idxr], r1, sem1)
                d2 = pltpu.async_copy(bc_hbm.at[idxc], r2, sem2)
                d1.wait()
                d2.wait()
                pltpu.sync_copy(r1, g1_hbm.at[pl.ds(off, _CH)])
                pltpu.sync_copy(r2, g2_hbm.at[pl.ds(off, _CH)])

    return k(a_tab, bc_tab, row, col)


def _edge_mlp(g1, g2, edge_attr, wae, we2, wn1b, be1, be2, bn1w):
    be = 2000

    def body(g1_ref, g2_ref, ea_ref, wae_ref, we2_ref, wn1b_ref,
             be1_ref, be2_ref, bn1w_ref, h2_ref, wc_s, bc_s):
        @pl.when(pl.program_id(0) == 0)
        def _():
            wc_s[...] = jnp.dot(we2_ref[...], wn1b_ref[...],
                                preferred_element_type=jnp.float32)
            bc_s[...] = jnp.dot(be2_ref[...], wn1b_ref[...],
                                preferred_element_type=jnp.float32) + bn1w_ref[...]

        xa = jnp.concatenate([g1_ref[...], ea_ref[...]], axis=1)
        h1 = jnp.maximum(
            g2_ref[:, :_D2]
            + jnp.dot(xa, wae_ref[...], preferred_element_type=jnp.float32)
            + be1_ref[...], 0.0)
        h2 = jnp.maximum(
            g2_ref[:, _D2:] + jnp.dot(h1, wc_s[...],
                                      preferred_element_type=jnp.float32)
            + bc_s[...], 0.0)
        lane = lax.broadcasted_iota(jnp.int32, (be, _D1), 1)
        h2_ref[...] = jnp.concatenate(
            [h2, (lane[:, _D2:] == _D2).astype(jnp.float32)], axis=1)

    return pl.pallas_call(
        body,
        grid=(_E // be,),
        in_specs=[pl.BlockSpec((be, _D1), lambda i: (i, 0)),
                  pl.BlockSpec((be, _D1), lambda i: (i, 0)),
                  pl.BlockSpec((be, _D1), lambda i: (i, 0)),
                  pl.BlockSpec((2 * _D1, _D2), lambda i: (0, 0)),
                  pl.BlockSpec((_D2, _D3), lambda i: (0, 0)),
                  pl.BlockSpec((_D3, _D2), lambda i: (0, 0)),
                  pl.BlockSpec((1, _D2), lambda i: (0, 0)),
                  pl.BlockSpec((1, _D3), lambda i: (0, 0)),
                  pl.BlockSpec((1, _D2), lambda i: (0, 0))],
        out_specs=pl.BlockSpec((be, _D1), lambda i: (i, 0)),
        out_shape=jax.ShapeDtypeStruct((_E, _D1), jnp.float32),
        scratch_shapes=[pltpu.VMEM((_D2, _D2), jnp.float32),
                        pltpu.VMEM((1, _D2), jnp.float32)],
    )(g1, g2, edge_attr, wae, we2, wn1b, be1, be2, bn1w)


def _sc_scatter(h2, row, zs):
    rpt = _NP // _NS

    @functools.partial(
        pl.kernel,
        out_type=jax.ShapeDtypeStruct((_NC, _NP, _D1), jnp.float32),
        mesh=_sc_mesh(),
        scratch_types=[
            pltpu.VMEM((_CH,), jnp.int32),
            pltpu.VMEM((_CH, _D1), jnp.float32),
            pltpu.VMEM_SHARED((_NP, _D1), jnp.float32),
        ],
    )
    def k(h2_hbm, row_hbm, zs_hbm, sp_hbm, idx, hv, s_sh):
        c = lax.axis_index("c")
        s = lax.axis_index("s")
        wid = s * _NC + c
        pltpu.sync_copy(zs_hbm.at[pl.ds(s * rpt, rpt)], s_sh.at[pl.ds(s * rpt, rpt)])
        plsc.subcore_barrier()

        @pl.loop(0, _TRIPS)
        def _trip(t):
            cidx = wid + t * _NW

            @pl.when(cidx < _NCHUNK)
            def _():
                off = cidx * _CH
                pltpu.sync_copy(row_hbm.at[pl.ds(off, _CH)], idx)
                pltpu.sync_copy(h2_hbm.at[pl.ds(off, _CH)], hv)
                pltpu.sync_copy(hv, s_sh.at[idx], add=True)

        plsc.subcore_barrier()
        pltpu.sync_copy(s_sh.at[pl.ds(s * rpt, rpt)],
                        sp_hbm.at[c, pl.ds(s * rpt, rpt)])

    return k(h2, row, zs)


def _final(sparts, batch3, u, wn2, bn2w, wg1u, wg1g, bg1, wg2, bg2,
           wfc1, bfc1, gamma, beta, wfc2p, bfc2p):
    bf = 1000
    nblk = _N // bf

    def body(sp_ref, b_ref, u_ref, wn2_ref, bn2w_ref, wg1u_ref,
             wg1g_ref, bg1_ref, wg2_ref, bg2_ref, wfc1_ref, bfc1_ref,
             gam_ref, bet_ref, wfc2_ref, bfc2_ref, out_ref,
             accr, accm, acca):
        i = pl.program_id(0)

        @pl.when(i == 0)
        def _():
            accr[...] = jnp.zeros_like(accr)
            accm[...] = jnp.zeros_like(accm)
            acca[...] = jnp.zeros_like(acca)

        st = sp_ref[0] + sp_ref[1]
        s = st[:, :_D2]
        cnt = st[:, _D2:_D2 + 1]
        mh = s / jnp.maximum(cnt, 1.0)
        nz = (cnt > 0.0).astype(jnp.float32)
        x2 = jnp.dot(mh, wn2_ref[...], preferred_element_type=jnp.float32) \
            + nz * bn2w_ref[...]
        r = jnp.maximum(x2, 0.0)
        bvals = b_ref[0, 0, :]
        onehot = (bvals[None, :]
                  == lax.broadcasted_iota(jnp.int32, (_B, bf), 0)
                  ).astype(jnp.float32)
        accr[...] += jnp.dot(onehot, r, preferred_element_type=jnp.float32)
        accm[...] += jnp.dot(onehot, mh, preferred_element_type=jnp.float32)
        aux = jnp.concatenate(
            [jnp.ones((bf, _D2), jnp.float32),
             jnp.broadcast_to(nz, (bf, _D2))], axis=1)
        acca[...] += jnp.dot(onehot, aux, preferred_element_type=jnp.float32)

        @pl.when(i == nblk - 1)
        def _():
            nb = acca[:, 0:1]
            nzc = acca[:, _D2:_D2 + 1]
            gp = (jnp.dot(accm[...], wn2_ref[...],
                          preferred_element_type=jnp.float32)
                  + nzc * bn2w_ref[...]) / jnp.maximum(nb, 1.0)
            g1h = jnp.maximum(
                u_ref[...] * wg1u_ref[...]
                + jnp.dot(gp, wg1g_ref[...], preferred_element_type=jnp.float32)
                + bg1_ref[...], 0.0)
            u2 = jnp.dot(g1h, wg2_ref[...],
                         preferred_element_type=jnp.float32) + bg2_ref[...]
            pooled = (accr[...] + jnp.maximum(u2, 0.0)) / (nb + 1.0)
            h = jnp.dot(pooled, wfc1_ref[...],
                        preferred_element_type=jnp.float32) + bfc1_ref[...]
            h = h * (1.0 / jnp.sqrt(1.0 + 1e-5)) * gam_ref[...] + bet_ref[...]
            h = jnp.maximum(h, 0.0)
            logits = jnp.dot(h, wfc2_ref[...],
                             preferred_element_type=jnp.float32) + bfc2_ref[...]
            colmask = lax.broadcasted_iota(jnp.int32, (_B, 128), 1) < 6
            lm = jnp.where(colmask, logits, -1e30)
            mx = jnp.max(lm, axis=1, keepdims=True)
            lse = jnp.log(jnp.sum(jnp.exp(lm - mx), axis=1, keepdims=True)) + mx
            out_ref[...] = lm - lse

    return pl.pallas_call(
        body,
        grid=(nblk,),
        in_specs=[pl.BlockSpec((_NC, bf, _D1), lambda i: (0, i, 0)),
                  pl.BlockSpec((1, 1, bf), lambda i: (i, 0, 0)),
                  pl.BlockSpec((_B, 1), lambda i: (0, 0)),
                  pl.BlockSpec((_D2, _D3), lambda i: (0, 0)),
                  pl.BlockSpec((1, _D3), lambda i: (0, 0)),
                  pl.BlockSpec((1, _D2), lambda i: (0, 0)),
                  pl.BlockSpec((_D3, _D2), lambda i: (0, 0)),
                  pl.BlockSpec((1, _D2), lambda i: (0, 0)),
                  pl.BlockSpec((_D2, _D3), lambda i: (0, 0)),
                  pl.BlockSpec((1, _D3), lambda i: (0, 0)),
                  pl.BlockSpec((_D3, _D2), lambda i: (0, 0)),
                  pl.BlockSpec((1, _D2), lambda i: (0, 0)),
                  pl.BlockSpec((1, _D2), lambda i: (0, 0)),
                  pl.BlockSpec((1, _D2), lambda i: (0, 0)),
                  pl.BlockSpec((_D2, 128), lambda i: (0, 0)),
                  pl.BlockSpec((1, 128), lambda i: (0, 0))],
        out_specs=pl.BlockSpec((_B, 128), lambda i: (0, 0)),
        out_shape=jax.ShapeDtypeStruct((_B, 128), jnp.float32),
        scratch_shapes=[pltpu.VMEM((_B, _D3), jnp.float32),
                        pltpu.VMEM((_B, _D2), jnp.float32),
                        pltpu.VMEM((_B, 128), jnp.float32)],
    )(sparts, batch3, u, wn2, bn2w, wg1u, wg1g, bg1, wg2, bg2,
      wfc1, bfc1, gamma, beta, wfc2p, bfc2p)


def kernel(x, edge_index, edge_attr, u, batch, We1, be1, We2, be2, Wn1, bn1w,
           Wn2, bn2w, Wg1, bg1, Wg2, bg2, Wfc1, bfc1, gamma, beta, Wfc2, bfc2):
    row = edge_index[0]
    col = edge_index[1]
    wcat = jnp.concatenate([We1[_D1:2 * _D1], Wn1[:_D1]], axis=1)
    wae = jnp.concatenate([We1[:_D1], We1[2 * _D1:]], axis=0)

    bc_tab = _nodeproj(x, wcat)
    g1, g2 = _sc_gather(x, bc_tab, row, col)
    h2 = _edge_mlp(g1, g2, edge_attr, wae, We2, Wn1[_D1:],
                   be1.reshape(1, _D2), be2.reshape(1, _D3),
                   bn1w.reshape(1, _D2))
    zs = jnp.zeros((_NP, _D1), jnp.float32)
    sparts = _sc_scatter(h2, row, zs)

    batch3 = batch.reshape(_N // 1000, 1, 1000)
    out = _final(sparts, batch3, u, Wn2, bn2w.reshape(1, _D3),
                 Wg1[0:1], Wg1[1:], bg1.reshape(1, _D2), Wg2,
                 bg2.reshape(1, _D3), Wfc1, bfc1.reshape(1, _D2),
                 gamma.reshape(1, _D2), beta.reshape(1, _D2),
                 jnp.pad(Wfc2, ((0, 0), (0, 122))),
                 jnp.pad(bfc2, (0, 122)).reshape(1, 128))
    return out[:, :6]


# speedup vs baseline: 6.7989x; 1.0269x over previous
import functools

import jax
import jax.numpy as jnp
from jax import lax
from jax.experimental import pallas as pl
from jax.experimental.pallas import tpu as pltpu
from jax.experimental.pallas import tpu_sc as plsc

_N = 10000
_NP = 10240
_E = 160000
_B = 8
_D1 = 128
_D2 = 64
_D3 = 640

_NC = 2
_NS = 16
_NW = _NC * _NS
_CH = 128
_NSL = 5
_ES = _E // _NSL
_SCHUNK = _ES // _CH
_STRIPS = -(-_SCHUNK // _NW)


def _sc_mesh():
    return plsc.VectorSubcoreMesh(core_axis_name="c", subcore_axis_name="s",
                                  num_cores=_NC, num_subcores=_NS)


def _nodeproj(x, wcat):
    bn = 2000

    def body(x_ref, w_ref, bc_ref):
        bc_ref[...] = jnp.dot(x_ref[...], w_ref[...],
                              preferred_element_type=jnp.float32)

    return pl.pallas_call(
        body,
        grid=(_N // bn,),
        in_specs=[pl.BlockSpec((bn, _D1), lambda i: (i, 0)),
                  pl.BlockSpec((_D1, _D1), lambda i: (0, 0))],
        out_specs=pl.BlockSpec((bn, _D1), lambda i: (i, 0)),
        out_shape=jax.ShapeDtypeStruct((_N, _D1), jnp.float32),
    )(x, wcat)


def _sc_gather(a_tab, bc_tab, row, col):
    ne = row.shape[0]
    nchunk = ne // _CH
    trips = -(-nchunk // _NW)

    @functools.partial(
        pl.kernel,
        out_type=[jax.ShapeDtypeStruct((ne, _D1), jnp.float32),
                  jax.ShapeDtypeStruct((ne, _D1), jnp.float32)],
        mesh=_sc_mesh(),
        scratch_types=[
            pltpu.VMEM((_CH,), jnp.int32),
            pltpu.VMEM((_CH,), jnp.int32),
            pltpu.VMEM((_CH, _D1), jnp.float32),
            pltpu.VMEM((_CH, _D1), jnp.float32),
            pltpu.SemaphoreType.DMA,
            pltpu.SemaphoreType.DMA,
        ],
    )
    def k(a_hbm, bc_hbm, row_hbm, col_hbm, g1_hbm, g2_hbm,
          idxr, idxc, r1, r2, sem1, sem2):
        wid = lax.axis_index("s") * _NC + lax.axis_index("c")

        @pl.loop(0, trips)
        def _trip(t):
            cidx = wid + t * _NW

            @pl.when(cidx < nchunk)
            def _():
                off = cidx * _CH
                pltpu.sync_copy(row_hbm.at[pl.ds(off, _CH)], idxr)
                pltpu.sync_copy(col_hbm.at[pl.ds(off, _CH)], idxc)
                d1 = pltpu.async_copy(a_hbm.at[idxr], r1, sem1)
                d2 = pltpu.async_copy(bc_hbm.at[idxc], r2, sem2)
                d1.wait()
                d2.wait()
                pltpu.sync_copy(r1, g1_hbm.at[pl.ds(off, _CH)])
                pltpu.sync_copy(r2, g2_hbm.at[pl.ds(off, _CH)])

    return k(a_tab, bc_tab, row, col)


def _edge_mlp(g1, g2, edge_attr, wae, we2, wn1b, be1, be2, bn1w):
    be = 2000

    def body(g1_ref, g2_ref, ea_ref, wae_ref, we2_ref, wn1b_ref,
             be1_ref, be2_ref, bn1w_ref, h2_ref, wc_s, bc_s):
        @pl.when(pl.program_id(0) == 0)
        def _():
            wc_s[...] = jnp.dot(we2_ref[...], wn1b_ref[...],
                                preferred_element_type=jnp.float32)
            bc_s[...] = jnp.dot(be2_ref[...], wn1b_ref[...],
                                preferred_element_type=jnp.float32) + bn1w_ref[...]

        xa = jnp.concatenate([g1_ref[...], ea_ref[...]], axis=1)
        h1 = jnp.maximum(
            g2_ref[:, :_D2]
            + jnp.dot(xa, wae_ref[...], preferred_element_type=jnp.float32)
            + be1_ref[...], 0.0)
        h2 = jnp.maximum(
            g2_ref[:, _D2:] + jnp.dot(h1, wc_s[...],
                                      preferred_element_type=jnp.float32)
            + bc_s[...], 0.0)
        lane = lax.broadcasted_iota(jnp.int32, (be, _D1), 1)
        h2_ref[...] = jnp.concatenate(
            [h2, (lane[:, _D2:] == _D2).astype(jnp.float32)], axis=1)

    return pl.pallas_call(
        body,
        grid=(g1.shape[0] // be,),
        in_specs=[pl.BlockSpec((be, _D1), lambda i: (i, 0)),
                  pl.BlockSpec((be, _D1), lambda i: (i, 0)),
                  pl.BlockSpec((be, _D1), lambda i: (i, 0)),
                  pl.BlockSpec((2 * _D1, _D2), lambda i: (0, 0)),
                  pl.BlockSpec((_D2, _D3), lambda i: (0, 0)),
                  pl.BlockSpec((_D3, _D2), lambda i: (0, 0)),
                  pl.BlockSpec((1, _D2), lambda i: (0, 0)),
                  pl.BlockSpec((1, _D3), lambda i: (0, 0)),
                  pl.BlockSpec((1, _D2), lambda i: (0, 0))],
        out_specs=pl.BlockSpec((be, _D1), lambda i: (i, 0)),
        out_shape=jax.ShapeDtypeStruct((g1.shape[0], _D1), jnp.float32),
        scratch_shapes=[pltpu.VMEM((_D2, _D2), jnp.float32),
                        pltpu.VMEM((1, _D2), jnp.float32)],
    )(g1, g2, edge_attr, wae, we2, wn1b, be1, be2, bn1w)


def _sc_scatter(h2s, rows, zs):
    rpt = _NP // _NS

    @functools.partial(
        pl.kernel,
        out_type=jax.ShapeDtypeStruct((_NC, _NP, _D1), jnp.float32),
        mesh=_sc_mesh(),
        scratch_types=[
            pltpu.VMEM((_CH,), jnp.int32),
            pltpu.VMEM((_CH, _D1), jnp.float32),
            pltpu.VMEM_SHARED((_NP, _D1), jnp.float32),
        ],
    )
    def k(*refs):
        h2_hbms = refs[0:_NSL]
        row_hbms = refs[_NSL:2 * _NSL]
        zs_hbm = refs[2 * _NSL]
        sp_hbm = refs[2 * _NSL + 1]
        idx, hv, s_sh = refs[2 * _NSL + 2:]
        c = lax.axis_index("c")
        s = lax.axis_index("s")
        wid = s * _NC + c
        pltpu.sync_copy(zs_hbm.at[pl.ds(s * rpt, rpt)], s_sh.at[pl.ds(s * rpt, rpt)])
        plsc.subcore_barrier()

        for sub in range(_NSL):
            h2_hbm = h2_hbms[sub]
            row_hbm = row_hbms[sub]

            @pl.loop(0, _STRIPS)
            def _trip(t, h2_hbm=h2_hbm, row_hbm=row_hbm):
                cidx = wid + t * _NW

                @pl.when(cidx < _SCHUNK)
                def _():
                    off = cidx * _CH
                    pltpu.sync_copy(row_hbm.at[pl.ds(off, _CH)], idx)
                    pltpu.sync_copy(h2_hbm.at[pl.ds(off, _CH)], hv)
                    pltpu.sync_copy(hv, s_sh.at[idx], add=True)

        plsc.subcore_barrier()
        pltpu.sync_copy(s_sh.at[pl.ds(s * rpt, rpt)],
                        sp_hbm.at[c, pl.ds(s * rpt, rpt)])

    return k(*h2s, *rows, zs)


def _final(sparts, batch3, u, wn2, bn2w, wg1u, wg1g, bg1, wg2, bg2,
           wfc1, bfc1, gamma, beta, wfc2p, bfc2p):
    bf = 1000
    nblk = _N // bf

    def body(sp_ref, b_ref, u_ref, wn2_ref, bn2w_ref, wg1u_ref,
             wg1g_ref, bg1_ref, wg2_ref, bg2_ref, wfc1_ref, bfc1_ref,
             gam_ref, bet_ref, wfc2_ref, bfc2_ref, out_ref,
             accr, accm, acca):
        i = pl.program_id(0)

        @pl.when(i == 0)
        def _():
            accr[...] = jnp.zeros_like(accr)
            accm[...] = jnp.zeros_like(accm)
            acca[...] = jnp.zeros_like(acca)

        st = sp_ref[0] + sp_ref[1]
        s = st[:, :_D2]
        cnt = st[:, _D2:_D2 + 1]
        mh = s / jnp.maximum(cnt, 1.0)
        nz = (cnt > 0.0).astype(jnp.float32)
        x2 = jnp.dot(mh, wn2_ref[...], preferred_element_type=jnp.float32) \
            + nz * bn2w_ref[...]
        r = jnp.maximum(x2, 0.0)
        bvals = b_ref[0, 0, :]
        onehot = (bvals[None, :]
                  == lax.broadcasted_iota(jnp.int32, (_B, bf), 0)
                  ).astype(jnp.float32)
        accr[...] += jnp.dot(onehot, r, preferred_element_type=jnp.float32)
        accm[...] += jnp.dot(onehot, mh, preferred_element_type=jnp.float32)
        aux = jnp.concatenate(
            [jnp.ones((bf, _D2), jnp.float32),
             jnp.broadcast_to(nz, (bf, _D2))], axis=1)
        acca[...] += jnp.dot(onehot, aux, preferred_element_type=jnp.float32)

        @pl.when(i == nblk - 1)
        def _():
            nb = acca[:, 0:1]
            nzc = acca[:, _D2:_D2 + 1]
            gp = (jnp.dot(accm[...], wn2_ref[...],
                          preferred_element_type=jnp.float32)
                  + nzc * bn2w_ref[...]) / jnp.maximum(nb, 1.0)
            g1h = jnp.maximum(
                u_ref[...] * wg1u_ref[...]
                + jnp.dot(gp, wg1g_ref[...], preferred_element_type=jnp.float32)
                + bg1_ref[...], 0.0)
            u2 = jnp.dot(g1h, wg2_ref[...],
                         preferred_element_type=jnp.float32) + bg2_ref[...]
            pooled = (accr[...] + jnp.maximum(u2, 0.0)) / (nb + 1.0)
            h = jnp.dot(pooled, wfc1_ref[...],
                        preferred_element_type=jnp.float32) + bfc1_ref[...]
            h = h * (1.0 / jnp.sqrt(1.0 + 1e-5)) * gam_ref[...] + bet_ref[...]
            h = jnp.maximum(h, 0.0)
            logits = jnp.dot(h, wfc2_ref[...],
                             preferred_element_type=jnp.float32) + bfc2_ref[...]
            colmask = lax.broadcasted_iota(jnp.int32, (_B, 128), 1) < 6
            lm = jnp.where(colmask, logits, -1e30)
            mx = jnp.max(lm, axis=1, keepdims=True)
            lse = jnp.log(jnp.sum(jnp.exp(lm - mx), axis=1, keepdims=True)) + mx
            out_ref[...] = lm - lse

    return pl.pallas_call(
        body,
        grid=(nblk,),
        in_specs=[pl.BlockSpec((_NC, bf, _D1), lambda i: (0, i, 0)),
                  pl.BlockSpec((1, 1, bf), lambda i: (i, 0, 0)),
                  pl.BlockSpec((_B, 1), lambda i: (0, 0)),
                  pl.BlockSpec((_D2, _D3), lambda i: (0, 0)),
                  pl.BlockSpec((1, _D3), lambda i: (0, 0)),
                  pl.BlockSpec((1, _D2), lambda i: (0, 0)),
                  pl.BlockSpec((_D3, _D2), lambda i: (0, 0)),
                  pl.BlockSpec((1, _D2), lambda i: (0, 0)),
                  pl.BlockSpec((_D2, _D3), lambda i: (0, 0)),
                  pl.BlockSpec((1, _D3), lambda i: (0, 0)),
                  pl.BlockSpec((_D3, _D2), lambda i: (0, 0)),
                  pl.BlockSpec((1, _D2), lambda i: (0, 0)),
                  pl.BlockSpec((1, _D2), lambda i: (0, 0)),
                  pl.BlockSpec((1, _D2), lambda i: (0, 0)),
                  pl.BlockSpec((_D2, 128), lambda i: (0, 0)),
                  pl.BlockSpec((1, 128), lambda i: (0, 0))],
        out_specs=pl.BlockSpec((_B, 128), lambda i: (0, 0)),
        out_shape=jax.ShapeDtypeStruct((_B, 128), jnp.float32),
        scratch_shapes=[pltpu.VMEM((_B, _D3), jnp.float32),
                        pltpu.VMEM((_B, _D2), jnp.float32),
                        pltpu.VMEM((_B, 128), jnp.float32)],
    )(sparts, batch3, u, wn2, bn2w, wg1u, wg1g, bg1, wg2, bg2,
      wfc1, bfc1, gamma, beta, wfc2p, bfc2p)


def kernel(x, edge_index, edge_attr, u, batch, We1, be1, We2, be2, Wn1, bn1w,
           Wn2, bn2w, Wg1, bg1, Wg2, bg2, Wfc1, bfc1, gamma, beta, Wfc2, bfc2):
    row = edge_index[0]
    col = edge_index[1]
    wcat = jnp.concatenate([We1[_D1:2 * _D1], Wn1[:_D1]], axis=1)
    wae = jnp.concatenate([We1[:_D1], We1[2 * _D1:]], axis=0)

    bc_tab = _nodeproj(x, wcat)
    h2s, rows = [], []
    for c in range(_NSL):
        rc = lax.slice_in_dim(row, c * _ES, (c + 1) * _ES)
        cc = lax.slice_in_dim(col, c * _ES, (c + 1) * _ES)
        g1, g2 = _sc_gather(x, bc_tab, rc, cc)
        eac = lax.slice_in_dim(edge_attr, c * _ES, (c + 1) * _ES)
        h2s.append(_edge_mlp(g1, g2, eac, wae, We2, Wn1[_D1:],
                             be1.reshape(1, _D2), be2.reshape(1, _D3),
                             bn1w.reshape(1, _D2)))
        rows.append(rc)
    zs = jnp.zeros((_NP, _D1), jnp.float32)
    sparts = _sc_scatter(h2s, rows, zs)

    batch3 = batch.reshape(_N // 1000, 1, 1000)
    out = _final(sparts, batch3, u, Wn2, bn2w.reshape(1, _D3),
                 Wg1[0:1], Wg1[1:], bg1.reshape(1, _D2), Wg2,
                 bg2.reshape(1, _D3), Wfc1, bfc1.reshape(1, _D2),
                 gamma.reshape(1, _D2), beta.reshape(1, _D2),
                 jnp.pad(Wfc2, ((0, 0), (0, 122))),
                 jnp.pad(bfc2, (0, 122)).reshape(1, 128))
    return out[:, :6]


# speedup vs baseline: 7.5130x; 1.1050x over previous
import functools

import jax
import jax.numpy as jnp
from jax import lax
from jax.experimental import pallas as pl
from jax.experimental.pallas import tpu as pltpu
from jax.experimental.pallas import tpu_sc as plsc

_N = 10000
_NP = 10240
_E = 160000
_B = 8
_D1 = 128
_D2 = 64
_D3 = 640

_NC = 2
_NS = 16
_NW = _NC * _NS
_CH = 128
_NSL = 5
_ES = _E // _NSL
_SCHUNK = _ES // _CH
_STRIPS = -(-_SCHUNK // _NW)


def _sc_mesh():
    return plsc.VectorSubcoreMesh(core_axis_name="c", subcore_axis_name="s",
                                  num_cores=_NC, num_subcores=_NS)


def _nodeproj(x, wcat):
    bn = 2000

    def body(x_ref, w_ref, bc_ref):
        bc_ref[...] = jnp.dot(x_ref[...], w_ref[...],
                              preferred_element_type=jnp.float32)

    return pl.pallas_call(
        body,
        grid=(_N // bn,),
        in_specs=[pl.BlockSpec((bn, _D1), lambda i: (i, 0)),
                  pl.BlockSpec((_D1, _D1), lambda i: (0, 0))],
        out_specs=pl.BlockSpec((bn, _D1), lambda i: (i, 0)),
        out_shape=jax.ShapeDtypeStruct((_N, _D1), jnp.float32),
    )(x, wcat)


def _sc_gather(a_tab, bc_tab, row, col):
    ne = row.shape[0]
    nchunk = ne // _CH
    trips = -(-nchunk // _NW)

    @functools.partial(
        pl.kernel,
        out_type=[jax.ShapeDtypeStruct((ne, _D1), jnp.float32),
                  jax.ShapeDtypeStruct((ne, _D1), jnp.float32)],
        mesh=_sc_mesh(),
        scratch_types=[
            pltpu.VMEM((2, _CH), jnp.int32),
            pltpu.VMEM((2, _CH), jnp.int32),
            pltpu.VMEM((2, _CH, _D1), jnp.float32),
            pltpu.VMEM((2, _CH, _D1), jnp.float32),
            pltpu.SemaphoreType.DMA,
            pltpu.SemaphoreType.DMA,
        ],
    )
    def k(a_hbm, bc_hbm, row_hbm, col_hbm, g1_hbm, g2_hbm,
          idxr, idxc, r1, r2, sg0, sg1):
        wid = lax.axis_index("s") * _NC + lax.axis_index("c")
        sems = (sg0, sg1)

        def fetch(t, b):
            off = (wid + t * _NW) * _CH
            pltpu.sync_copy(row_hbm.at[pl.ds(off, _CH)], idxr.at[b])
            pltpu.sync_copy(col_hbm.at[pl.ds(off, _CH)], idxc.at[b])
            pltpu.async_copy(a_hbm.at[idxr.at[b]], r1.at[b], sems[b])
            pltpu.async_copy(bc_hbm.at[idxc.at[b]], r2.at[b], sems[b])

        def drain_wb(t, b):
            off = (wid + t * _NW) * _CH
            pltpu.make_async_copy(a_hbm.at[idxr.at[b]], r1.at[b], sems[b]).wait()
            pltpu.make_async_copy(bc_hbm.at[idxc.at[b]], r2.at[b], sems[b]).wait()
            pltpu.sync_copy(r1.at[b], g1_hbm.at[pl.ds(off, _CH)])
            pltpu.sync_copy(r2.at[b], g2_hbm.at[pl.ds(off, _CH)])

        fetch(0, 0)

        @pl.loop(0, trips, step=2)
        def _outer(t0):
            for b in (0, 1):
                t = t0 + b

                @pl.when(wid + (t + 1) * _NW < nchunk)
                def _(t=t, b=b):
                    fetch(t + 1, 1 - b)

                @pl.when(wid + t * _NW < nchunk)
                def _(t=t, b=b):
                    drain_wb(t, b)

    return k(a_tab, bc_tab, row, col)


def _edge_mlp(g1, g2, edge_attr, wae, we2, wn1b, be1, be2, bn1w):
    be = 2000

    def body(g1_ref, g2_ref, ea_ref, wae_ref, we2_ref, wn1b_ref,
             be1_ref, be2_ref, bn1w_ref, h2_ref, wc_s, bc_s):
        @pl.when(pl.program_id(0) == 0)
        def _():
            wc_s[...] = jnp.dot(we2_ref[...], wn1b_ref[...],
                                preferred_element_type=jnp.float32)
            bc_s[...] = jnp.dot(be2_ref[...], wn1b_ref[...],
                                preferred_element_type=jnp.float32) + bn1w_ref[...]

        xa = jnp.concatenate([g1_ref[...], ea_ref[...]], axis=1)
        h1 = jnp.maximum(
            g2_ref[:, :_D2]
            + jnp.dot(xa, wae_ref[...], preferred_element_type=jnp.float32)
            + be1_ref[...], 0.0)
        h2 = jnp.maximum(
            g2_ref[:, _D2:] + jnp.dot(h1, wc_s[...],
                                      preferred_element_type=jnp.float32)
            + bc_s[...], 0.0)
        lane = lax.broadcasted_iota(jnp.int32, (be, _D1), 1)
        h2_ref[...] = jnp.concatenate(
            [h2, (lane[:, _D2:] == _D2).astype(jnp.float32)], axis=1)

    return pl.pallas_call(
        body,
        grid=(g1.shape[0] // be,),
        in_specs=[pl.BlockSpec((be, _D1), lambda i: (i, 0)),
                  pl.BlockSpec((be, _D1), lambda i: (i, 0)),
                  pl.BlockSpec((be, _D1), lambda i: (i, 0)),
                  pl.BlockSpec((2 * _D1, _D2), lambda i: (0, 0)),
                  pl.BlockSpec((_D2, _D3), lambda i: (0, 0)),
                  pl.BlockSpec((_D3, _D2), lambda i: (0, 0)),
                  pl.BlockSpec((1, _D2), lambda i: (0, 0)),
                  pl.BlockSpec((1, _D3), lambda i: (0, 0)),
                  pl.BlockSpec((1, _D2), lambda i: (0, 0))],
        out_specs=pl.BlockSpec((be, _D1), lambda i: (i, 0)),
        out_shape=jax.ShapeDtypeStruct((g1.shape[0], _D1), jnp.float32),
        scratch_shapes=[pltpu.VMEM((_D2, _D2), jnp.float32),
                        pltpu.VMEM((1, _D2), jnp.float32)],
    )(g1, g2, edge_attr, wae, we2, wn1b, be1, be2, bn1w)


def _sc_scatter(h2s, rows, zs):
    rpt = _NP // _NS

    @functools.partial(
        pl.kernel,
        out_type=jax.ShapeDtypeStruct((_NC, _NP, _D1), jnp.float32),
        mesh=_sc_mesh(),
        scratch_types=[
            pltpu.VMEM((2, _CH), jnp.int32),
            pltpu.VMEM((2, _CH, _D1), jnp.float32),
            pltpu.VMEM_SHARED((_NP, _D1), jnp.float32),
            pltpu.SemaphoreType.DMA,
            pltpu.SemaphoreType.DMA,
        ],
    )
    def k(*refs):
        h2_hbms = refs[0:_NSL]
        row_hbms = refs[_NSL:2 * _NSL]
        zs_hbm = refs[2 * _NSL]
        sp_hbm = refs[2 * _NSL + 1]
        idx, hv, s_sh, sh0, sh1 = refs[2 * _NSL + 2:]
        c = lax.axis_index("c")
        s = lax.axis_index("s")
        wid = s * _NC + c
        sems = (sh0, sh1)
        pltpu.sync_copy(zs_hbm.at[pl.ds(s * rpt, rpt)], s_sh.at[pl.ds(s * rpt, rpt)])
        plsc.subcore_barrier()

        def fetch(sub, t, b):
            off = (wid + t * _NW) * _CH
            pltpu.sync_copy(row_hbms[sub].at[pl.ds(off, _CH)], idx.at[b])
            pltpu.async_copy(h2_hbms[sub].at[pl.ds(off, _CH)], hv.at[b], sems[b])

        def drain_add(sub, b):
            pltpu.make_async_copy(h2_hbms[sub].at[pl.ds(0, _CH)], hv.at[b],
                                  sems[b]).wait()
            pltpu.sync_copy(hv.at[b], s_sh.at[idx.at[b]], add=True)

        for sub in range(_NSL):
            fetch(sub, 0, 0)

            @pl.loop(0, _STRIPS, step=2)
            def _outer(t0, sub=sub):
                for b in (0, 1):
                    t = t0 + b

                    @pl.when(wid + (t + 1) * _NW < _SCHUNK)
                    def _(t=t, b=b, sub=sub):
                        fetch(sub, t + 1, 1 - b)

                    @pl.when(wid + t * _NW < _SCHUNK)
                    def _(t=t, b=b, sub=sub):
                        drain_add(sub, b)

        plsc.subcore_barrier()
        pltpu.sync_copy(s_sh.at[pl.ds(s * rpt, rpt)],
                        sp_hbm.at[c, pl.ds(s * rpt, rpt)])

    return k(*h2s, *rows, zs)


def _final(sparts, batch3, u, wn2, bn2w, wg1u, wg1g, bg1, wg2, bg2,
           wfc1, bfc1, gamma, beta, wfc2p, bfc2p):
    bf = 1000
    nblk = _N // bf

    def body(sp_ref, b_ref, u_ref, wn2_ref, bn2w_ref, wg1u_ref,
             wg1g_ref, bg1_ref, wg2_ref, bg2_ref, wfc1_ref, bfc1_ref,
             gam_ref, bet_ref, wfc2_ref, bfc2_ref, out_ref,
             accr, accm, acca):
        i = pl.program_id(0)

        @pl.when(i == 0)
        def _():
            accr[...] = jnp.zeros_like(accr)
            accm[...] = jnp.zeros_like(accm)
            acca[...] = jnp.zeros_like(acca)

        st = sp_ref[0] + sp_ref[1]
        s = st[:, :_D2]
        cnt = st[:, _D2:_D2 + 1]
        mh = s / jnp.maximum(cnt, 1.0)
        nz = (cnt > 0.0).astype(jnp.float32)
        x2 = jnp.dot(mh, wn2_ref[...], preferred_element_type=jnp.float32) \
            + nz * bn2w_ref[...]
        r = jnp.maximum(x2, 0.0)
        bvals = b_ref[0, 0, :]
        onehot = (bvals[None, :]
                  == lax.broadcasted_iota(jnp.int32, (_B, bf), 0)
                  ).astype(jnp.float32)
        accr[...] += jnp.dot(onehot, r, preferred_element_type=jnp.float32)
        accm[...] += jnp.dot(onehot, mh, preferred_element_type=jnp.float32)
        aux = jnp.concatenate(
            [jnp.ones((bf, _D2), jnp.float32),
             jnp.broadcast_to(nz, (bf, _D2))], axis=1)
        acca[...] += jnp.dot(onehot, aux, preferred_element_type=jnp.float32)

        @pl.when(i == nblk - 1)
        def _():
            nb = acca[:, 0:1]
            nzc = acca[:, _D2:_D2 + 1]
            gp = (jnp.dot(accm[...], wn2_ref[...],
                          preferred_element_type=jnp.float32)
                  + nzc * bn2w_ref[...]) / jnp.maximum(nb, 1.0)
            g1h = jnp.maximum(
                u_ref[...] * wg1u_ref[...]
                + jnp.dot(gp, wg1g_ref[...], preferred_element_type=jnp.float32)
                + bg1_ref[...], 0.0)
            u2 = jnp.dot(g1h, wg2_ref[...],
                         preferred_element_type=jnp.float32) + bg2_ref[...]
            pooled = (accr[...] + jnp.maximum(u2, 0.0)) / (nb + 1.0)
            h = jnp.dot(pooled, wfc1_ref[...],
                        preferred_element_type=jnp.float32) + bfc1_ref[...]
            h = h * (1.0 / jnp.sqrt(1.0 + 1e-5)) * gam_ref[...] + bet_ref[...]
            h = jnp.maximum(h, 0.0)
            logits = jnp.dot(h, wfc2_ref[...],
                             preferred_element_type=jnp.float32) + bfc2_ref[...]
            colmask = lax.broadcasted_iota(jnp.int32, (_B, 128), 1) < 6
            lm = jnp.where(colmask, logits, -1e30)
            mx = jnp.max(lm, axis=1, keepdims=True)
            lse = jnp.log(jnp.sum(jnp.exp(lm - mx), axis=1, keepdims=True)) + mx
            out_ref[...] = lm - lse

    return pl.pallas_call(
        body,
        grid=(nblk,),
        in_specs=[pl.BlockSpec((_NC, bf, _D1), lambda i: (0, i, 0)),
                  pl.BlockSpec((1, 1, bf), lambda i: (i, 0, 0)),
                  pl.BlockSpec((_B, 1), lambda i: (0, 0)),
                  pl.BlockSpec((_D2, _D3), lambda i: (0, 0)),
                  pl.BlockSpec((1, _D3), lambda i: (0, 0)),
                  pl.BlockSpec((1, _D2), lambda i: (0, 0)),
                  pl.BlockSpec((_D3, _D2), lambda i: (0, 0)),
                  pl.BlockSpec((1, _D2), lambda i: (0, 0)),
                  pl.BlockSpec((_D2, _D3), lambda i: (0, 0)),
                  pl.BlockSpec((1, _D3), lambda i: (0, 0)),
                  pl.BlockSpec((_D3, _D2), lambda i: (0, 0)),
                  pl.BlockSpec((1, _D2), lambda i: (0, 0)),
                  pl.BlockSpec((1, _D2), lambda i: (0, 0)),
                  pl.BlockSpec((1, _D2), lambda i: (0, 0)),
                  pl.BlockSpec((_D2, 128), lambda i: (0, 0)),
                  pl.BlockSpec((1, 128), lambda i: (0, 0))],
        out_specs=pl.BlockSpec((_B, 128), lambda i: (0, 0)),
        out_shape=jax.ShapeDtypeStruct((_B, 128), jnp.float32),
        scratch_shapes=[pltpu.VMEM((_B, _D3), jnp.float32),
                        pltpu.VMEM((_B, _D2), jnp.float32),
                        pltpu.VMEM((_B, 128), jnp.float32)],
    )(sparts, batch3, u, wn2, bn2w, wg1u, wg1g, bg1, wg2, bg2,
      wfc1, bfc1, gamma, beta, wfc2p, bfc2p)


def kernel(x, edge_index, edge_attr, u, batch, We1, be1, We2, be2, Wn1, bn1w,
           Wn2, bn2w, Wg1, bg1, Wg2, bg2, Wfc1, bfc1, gamma, beta, Wfc2, bfc2):
    row = edge_index[0]
    col = edge_index[1]
    wcat = jnp.concatenate([We1[_D1:2 * _D1], Wn1[:_D1]], axis=1)
    wae = jnp.concatenate([We1[:_D1], We1[2 * _D1:]], axis=0)

    bc_tab = _nodeproj(x, wcat)
    h2s, rows = [], []
    for c in range(_NSL):
        rc = lax.slice_in_dim(row, c * _ES, (c + 1) * _ES)
        cc = lax.slice_in_dim(col, c * _ES, (c + 1) * _ES)
        g1, g2 = _sc_gather(x, bc_tab, rc, cc)
        eac = lax.slice_in_dim(edge_attr, c * _ES, (c + 1) * _ES)
        h2s.append(_edge_mlp(g1, g2, eac, wae, We2, Wn1[_D1:],
                             be1.reshape(1, _D2), be2.reshape(1, _D3),
                             bn1w.reshape(1, _D2)))
        rows.append(rc)
    zs = jnp.zeros((_NP, _D1), jnp.float32)
    sparts = _sc_scatter(h2s, rows, zs)

    batch3 = batch.reshape(_N // 1000, 1, 1000)
    out = _final(sparts, batch3, u, Wn2, bn2w.reshape(1, _D3),
                 Wg1[0:1], Wg1[1:], bg1.reshape(1, _D2), Wg2,
                 bg2.reshape(1, _D3), Wfc1, bfc1.reshape(1, _D2),
                 gamma.reshape(1, _D2), beta.reshape(1, _D2),
                 jnp.pad(Wfc2, ((0, 0), (0, 122))),
                 jnp.pad(bfc2, (0, 122)).reshape(1, 128))
    return out[:, :6]


# speedup vs baseline: 8.4835x; 1.1292x over previous
import functools

import jax
import jax.numpy as jnp
from jax import lax
from jax.experimental import pallas as pl
from jax.experimental.pallas import tpu as pltpu
from jax.experimental.pallas import tpu_sc as plsc

_N = 10000
_NP = 10240
_E = 160000
_B = 8
_D1 = 128
_D2 = 64
_D3 = 640

_NC = 2
_NS = 16
_NW = _NC * _NS
_CH = 128
_NSL = 5
_ES = _E // _NSL
_SCHUNK = _ES // _CH
_STRIPS = -(-_SCHUNK // _NW)


def _sc_mesh():
    return plsc.VectorSubcoreMesh(core_axis_name="c", subcore_axis_name="s",
                                  num_cores=_NC, num_subcores=_NS)


def _nodeproj(x, wcat):
    bn = 2000

    def body(x_ref, w_ref, a_ref, bc_ref):
        p = jnp.dot(x_ref[...], w_ref[...], preferred_element_type=jnp.float32)
        a_ref[...] = jnp.concatenate(
            [p[:, :_D2], jnp.zeros((bn, _D2), jnp.float32)], axis=1)
        bc_ref[...] = p[:, _D2:]

    return pl.pallas_call(
        body,
        grid=(_N // bn,),
        in_specs=[pl.BlockSpec((bn, _D1), lambda i: (i, 0)),
                  pl.BlockSpec((_D1, 192), lambda i: (0, 0))],
        out_specs=[pl.BlockSpec((bn, _D1), lambda i: (i, 0)),
                   pl.BlockSpec((bn, _D1), lambda i: (i, 0))],
        out_shape=[jax.ShapeDtypeStruct((_N, _D1), jnp.float32),
                   jax.ShapeDtypeStruct((_N, _D1), jnp.float32)],
    )(x, wcat)


def _sc_gather(a_tab, bc_tab, row, col):
    ne = row.shape[0]
    nchunk = ne // _CH
    trips = -(-nchunk // _NW)

    @functools.partial(
        pl.kernel,
        out_type=jax.ShapeDtypeStruct((ne, _D1), jnp.float32),
        mesh=_sc_mesh(),
        scratch_types=[
            pltpu.VMEM((2, _CH), jnp.int32),
            pltpu.VMEM((2, _CH), jnp.int32),
            pltpu.VMEM((2, _CH, _D1), jnp.float32),
            pltpu.VMEM((2, _CH, _D1), jnp.float32),
            pltpu.SemaphoreType.DMA,
            pltpu.SemaphoreType.DMA,
        ],
    )
    def k(a_hbm, bc_hbm, row_hbm, col_hbm, g_hbm,
          idxr, idxc, r1, r2, sg0, sg1):
        wid = lax.axis_index("s") * _NC + lax.axis_index("c")
        sems = (sg0, sg1)

        def fetch(t, b):
            off = (wid + t * _NW) * _CH
            pltpu.sync_copy(row_hbm.at[pl.ds(off, _CH)], idxr.at[b])
            pltpu.sync_copy(col_hbm.at[pl.ds(off, _CH)], idxc.at[b])
            pltpu.async_copy(a_hbm.at[idxr.at[b]], r1.at[b], sems[b])
            pltpu.async_copy(bc_hbm.at[idxc.at[b]], r2.at[b], sems[b])

        def drain_wb(t, b):
            off = (wid + t * _NW) * _CH
            pltpu.make_async_copy(a_hbm.at[idxr.at[b]], r1.at[b], sems[b]).wait()
            pltpu.make_async_copy(bc_hbm.at[idxc.at[b]], r2.at[b], sems[b]).wait()

            @pl.loop(0, _CH, unroll=8)
            def _acc(e, b=b):
                for j in range(_D2 // 16):
                    r2[b, e, pl.ds(j * 16, 16)] = (
                        r2[b, e, pl.ds(j * 16, 16)]
                        + r1[b, e, pl.ds(j * 16, 16)])

            pltpu.sync_copy(r2.at[b], g_hbm.at[pl.ds(off, _CH)])

        fetch(0, 0)

        @pl.loop(0, trips, step=2)
        def _outer(t0):
            for b in (0, 1):
                t = t0 + b

                @pl.when(wid + (t + 1) * _NW < nchunk)
                def _(t=t, b=b):
                    fetch(t + 1, 1 - b)

                @pl.when(wid + t * _NW < nchunk)
                def _(t=t, b=b):
                    drain_wb(t, b)

    return k(a_tab, bc_tab, row, col)


def _edge_mlp(g, edge_attr, we1c, we2, wn1b, be1, be2, bn1w):
    be = 2000

    def body(g_ref, ea_ref, we1c_ref, we2_ref, wn1b_ref,
             be1_ref, be2_ref, bn1w_ref, h2_ref, wc_s, bc_s):
        @pl.when(pl.program_id(0) == 0)
        def _():
            wc_s[...] = jnp.dot(we2_ref[...], wn1b_ref[...],
                                preferred_element_type=jnp.float32)
            bc_s[...] = jnp.dot(be2_ref[...], wn1b_ref[...],
                                preferred_element_type=jnp.float32) + bn1w_ref[...]

        h1 = jnp.maximum(
            g_ref[:, :_D2]
            + jnp.dot(ea_ref[...], we1c_ref[...],
                      preferred_element_type=jnp.float32)
            + be1_ref[...], 0.0)
        h2 = jnp.maximum(
            g_ref[:, _D2:] + jnp.dot(h1, wc_s[...],
                                     preferred_element_type=jnp.float32)
            + bc_s[...], 0.0)
        lane = lax.broadcasted_iota(jnp.int32, (be, _D1), 1)
        h2_ref[...] = jnp.concatenate(
            [h2, (lane[:, _D2:] == _D2).astype(jnp.float32)], axis=1)

    return pl.pallas_call(
        body,
        grid=(g.shape[0] // be,),
        in_specs=[pl.BlockSpec((be, _D1), lambda i: (i, 0)),
                  pl.BlockSpec((be, _D1), lambda i: (i, 0)),
                  pl.BlockSpec((_D1, _D2), lambda i: (0, 0)),
                  pl.BlockSpec((_D2, _D3), lambda i: (0, 0)),
                  pl.BlockSpec((_D3, _D2), lambda i: (0, 0)),
                  pl.BlockSpec((1, _D2), lambda i: (0, 0)),
                  pl.BlockSpec((1, _D3), lambda i: (0, 0)),
                  pl.BlockSpec((1, _D2), lambda i: (0, 0))],
        out_specs=pl.BlockSpec((be, _D1), lambda i: (i, 0)),
        out_shape=jax.ShapeDtypeStruct((g.shape[0], _D1), jnp.float32),
        scratch_shapes=[pltpu.VMEM((_D2, _D2), jnp.float32),
                        pltpu.VMEM((1, _D2), jnp.float32)],
    )(g, edge_attr, we1c, we2, wn1b, be1, be2, bn1w)


def _sc_scatter(h2s, rows, zs):
    rpt = _NP // _NS

    @functools.partial(
        pl.kernel,
        out_type=jax.ShapeDtypeStruct((_NC, _NP, _D1), jnp.float32),
        mesh=_sc_mesh(),
        scratch_types=[
            pltpu.VMEM((2, _CH), jnp.int32),
            pltpu.VMEM((2, _CH, _D1), jnp.float32),
            pltpu.VMEM_SHARED((_NP, _D1), jnp.float32),
            pltpu.SemaphoreType.DMA,
            pltpu.SemaphoreType.DMA,
        ],
    )
    def k(*refs):
        h2_hbms = refs[0:_NSL]
        row_hbms = refs[_NSL:2 * _NSL]
        zs_hbm = refs[2 * _NSL]
        sp_hbm = refs[2 * _NSL + 1]
        idx, hv, s_sh, sh0, sh1 = refs[2 * _NSL + 2:]
        c = lax.axis_index("c")
        s = lax.axis_index("s")
        wid = s * _NC + c
        sems = (sh0, sh1)
        pltpu.sync_copy(zs_hbm.at[pl.ds(s * rpt, rpt)], s_sh.at[pl.ds(s * rpt, rpt)])
        plsc.subcore_barrier()

        def fetch(sub, t, b):
            off = (wid + t * _NW) * _CH
            pltpu.sync_copy(row_hbms[sub].at[pl.ds(off, _CH)], idx.at[b])
            pltpu.async_copy(h2_hbms[sub].at[pl.ds(off, _CH)], hv.at[b], sems[b])

        def drain_add(sub, b):
            pltpu.make_async_copy(h2_hbms[sub].at[pl.ds(0, _CH)], hv.at[b],
                                  sems[b]).wait()
            pltpu.sync_copy(hv.at[b], s_sh.at[idx.at[b]], add=True)

        for sub in range(_NSL):
            fetch(sub, 0, 0)

            @pl.loop(0, _STRIPS, step=2)
            def _outer(t0, sub=sub):
                for b in (0, 1):
                    t = t0 + b

                    @pl.when(wid + (t + 1) * _NW < _SCHUNK)
                    def _(t=t, b=b, sub=sub):
                        fetch(sub, t + 1, 1 - b)

                    @pl.when(wid + t * _NW < _SCHUNK)
                    def _(t=t, b=b, sub=sub):
                        drain_add(sub, b)

        plsc.subcore_barrier()
        pltpu.sync_copy(s_sh.at[pl.ds(s * rpt, rpt)],
                        sp_hbm.at[c, pl.ds(s * rpt, rpt)])

    return k(*h2s, *rows, zs)


def _final(sparts, batch3, u, wn2, bn2w, wg1u, wg1g, bg1, wg2, bg2,
           wfc1, bfc1, gamma, beta, wfc2p, bfc2p):
    bf = 1000
    nblk = _N // bf

    def body(sp_ref, b_ref, u_ref, wn2_ref, bn2w_ref, wg1u_ref,
             wg1g_ref, bg1_ref, wg2_ref, bg2_ref, wfc1_ref, bfc1_ref,
             gam_ref, bet_ref, wfc2_ref, bfc2_ref, out_ref,
             accr, accm, acca):
        i = pl.program_id(0)

        @pl.when(i == 0)
        def _():
            accr[...] = jnp.zeros_like(accr)
            accm[...] = jnp.zeros_like(accm)
            acca[...] = jnp.zeros_like(acca)

        st = sp_ref[0] + sp_ref[1]
        s = st[:, :_D2]
        cnt = st[:, _D2:_D2 + 1]
        mh = s / jnp.maximum(cnt, 1.0)
        nz = (cnt > 0.0).astype(jnp.float32)
        x2 = jnp.dot(mh, wn2_ref[...], preferred_element_type=jnp.float32) \
            + nz * bn2w_ref[...]
        r = jnp.maximum(x2, 0.0)
        bvals = b_ref[0, 0, :]
        onehot = (bvals[None, :]
                  == lax.broadcasted_iota(jnp.int32, (_B, bf), 0)
                  ).astype(jnp.float32)
        accr[...] += jnp.dot(onehot, r, preferred_element_type=jnp.float32)
        accm[...] += jnp.dot(onehot, mh, preferred_element_type=jnp.float32)
        aux = jnp.concatenate(
            [jnp.ones((bf, _D2), jnp.float32),
             jnp.broadcast_to(nz, (bf, _D2))], axis=1)
        acca[...] += jnp.dot(onehot, aux, preferred_element_type=jnp.float32)

        @pl.when(i == nblk - 1)
        def _():
            nb = acca[:, 0:1]
            nzc = acca[:, _D2:_D2 + 1]
            gp = (jnp.dot(accm[...], wn2_ref[...],
                          preferred_element_type=jnp.float32)
                  + nzc * bn2w_ref[...]) / jnp.maximum(nb, 1.0)
            g1h = jnp.maximum(
                u_ref[...] * wg1u_ref[...]
                + jnp.dot(gp, wg1g_ref[...], preferred_element_type=jnp.float32)
                + bg1_ref[...], 0.0)
            u2 = jnp.dot(g1h, wg2_ref[...],
                         preferred_element_type=jnp.float32) + bg2_ref[...]
            pooled = (accr[...] + jnp.maximum(u2, 0.0)) / (nb + 1.0)
            h = jnp.dot(pooled, wfc1_ref[...],
                        preferred_element_type=jnp.float32) + bfc1_ref[...]
            h = h * (1.0 / jnp.sqrt(1.0 + 1e-5)) * gam_ref[...] + bet_ref[...]
            h = jnp.maximum(h, 0.0)
            logits = jnp.dot(h, wfc2_ref[...],
                             preferred_element_type=jnp.float32) + bfc2_ref[...]
            colmask = lax.broadcasted_iota(jnp.int32, (_B, 128), 1) < 6
            lm = jnp.where(colmask, logits, -1e30)
            mx = jnp.max(lm, axis=1, keepdims=True)
            lse = jnp.log(jnp.sum(jnp.exp(lm - mx), axis=1, keepdims=True)) + mx
            out_ref[...] = lm - lse

    return pl.pallas_call(
        body,
        grid=(nblk,),
        in_specs=[pl.BlockSpec((_NC, bf, _D1), lambda i: (0, i, 0)),
                  pl.BlockSpec((1, 1, bf), lambda i: (i, 0, 0)),
                  pl.BlockSpec((_B, 1), lambda i: (0, 0)),
                  pl.BlockSpec((_D2, _D3), lambda i: (0, 0)),
                  pl.BlockSpec((1, _D3), lambda i: (0, 0)),
                  pl.BlockSpec((1, _D2), lambda i: (0, 0)),
                  pl.BlockSpec((_D3, _D2), lambda i: (0, 0)),
                  pl.BlockSpec((1, _D2), lambda i: (0, 0)),
                  pl.BlockSpec((_D2, _D3), lambda i: (0, 0)),
                  pl.BlockSpec((1, _D3), lambda i: (0, 0)),
                  pl.BlockSpec((_D3, _D2), lambda i: (0, 0)),
                  pl.BlockSpec((1, _D2), lambda i: (0, 0)),
                  pl.BlockSpec((1, _D2), lambda i: (0, 0)),
                  pl.BlockSpec((1, _D2), lambda i: (0, 0)),
                  pl.BlockSpec((_D2, 128), lambda i: (0, 0)),
                  pl.BlockSpec((1, 128), lambda i: (0, 0))],
        out_specs=pl.BlockSpec((_B, 128), lambda i: (0, 0)),
        out_shape=jax.ShapeDtypeStruct((_B, 128), jnp.float32),
        scratch_shapes=[pltpu.VMEM((_B, _D3), jnp.float32),
                        pltpu.VMEM((_B, _D2), jnp.float32),
                        pltpu.VMEM((_B, 128), jnp.float32)],
    )(sparts, batch3, u, wn2, bn2w, wg1u, wg1g, bg1, wg2, bg2,
      wfc1, bfc1, gamma, beta, wfc2p, bfc2p)


def kernel(x, edge_index, edge_attr, u, batch, We1, be1, We2, be2, Wn1, bn1w,
           Wn2, bn2w, Wg1, bg1, Wg2, bg2, Wfc1, bfc1, gamma, beta, Wfc2, bfc2):
    row = edge_index[0]
    col = edge_index[1]
    wcat = jnp.concatenate([We1[:_D1], We1[_D1:2 * _D1], Wn1[:_D1]], axis=1)

    a_tab, bc_tab = _nodeproj(x, wcat)
    h2s, rows = [], []
    for c in range(_NSL):
        rc = lax.slice_in_dim(row, c * _ES, (c + 1) * _ES)
        cc = lax.slice_in_dim(col, c * _ES, (c + 1) * _ES)
        g = _sc_gather(a_tab, bc_tab, rc, cc)
        eac = lax.slice_in_dim(edge_attr, c * _ES, (c + 1) * _ES)
        h2s.append(_edge_mlp(g, eac, We1[2 * _D1:], We2, Wn1[_D1:],
                             be1.reshape(1, _D2), be2.reshape(1, _D3),
                             bn1w.reshape(1, _D2)))
        rows.append(rc)
    zs = jnp.zeros((_NP, _D1), jnp.float32)
    sparts = _sc_scatter(h2s, rows, zs)

    batch3 = batch.reshape(_N // 1000, 1, 1000)
    out = _final(sparts, batch3, u, Wn2, bn2w.reshape(1, _D3),
                 Wg1[0:1], Wg1[1:], bg1.reshape(1, _D2), Wg2,
                 bg2.reshape(1, _D3), Wfc1, bfc1.reshape(1, _D2),
                 gamma.reshape(1, _D2), beta.reshape(1, _D2),
                 jnp.pad(Wfc2, ((0, 0), (0, 122))),
                 jnp.pad(bfc2, (0, 122)).reshape(1, 128))
    return out[:, :6]


# speedup vs baseline: 8.6078x; 1.0146x over previous
import functools

import jax
import jax.numpy as jnp
from jax import lax
from jax.experimental import pallas as pl
from jax.experimental.pallas import tpu as pltpu
from jax.experimental.pallas import tpu_sc as plsc

_N = 10000
_NP = 10240
_E = 160000
_B = 8
_D1 = 128
_D2 = 64
_D3 = 640

_NC = 2
_NS = 16
_NW = _NC * _NS
_CH = 128
_SLICES = (48000, 48000, 48000, 16000)
_NSL = len(_SLICES)


def _sc_mesh():
    return plsc.VectorSubcoreMesh(core_axis_name="c", subcore_axis_name="s",
                                  num_cores=_NC, num_subcores=_NS)


def _nodeproj(x, wcat):
    bn = 2000

    def body(x_ref, w_ref, a_ref, bc_ref):
        p = jnp.dot(x_ref[...], w_ref[...], preferred_element_type=jnp.float32)
        a_ref[...] = jnp.concatenate(
            [p[:, :_D2], jnp.zeros((bn, _D2), jnp.float32)], axis=1)
        bc_ref[...] = p[:, _D2:]

    return pl.pallas_call(
        body,
        grid=(_N // bn,),
        in_specs=[pl.BlockSpec((bn, _D1), lambda i: (i, 0)),
                  pl.BlockSpec((_D1, 192), lambda i: (0, 0))],
        out_specs=[pl.BlockSpec((bn, _D1), lambda i: (i, 0)),
                   pl.BlockSpec((bn, _D1), lambda i: (i, 0))],
        out_shape=[jax.ShapeDtypeStruct((_N, _D1), jnp.float32),
                   jax.ShapeDtypeStruct((_N, _D1), jnp.float32)],
    )(x, wcat)


def _sc_gather(a_tab, bc_tab, row, col):
    ne = row.shape[0]
    nchunk = ne // _CH
    trips = -(-nchunk // _NW)

    @functools.partial(
        pl.kernel,
        out_type=jax.ShapeDtypeStruct((ne, _D1), jnp.float32),
        mesh=_sc_mesh(),
        scratch_types=[
            pltpu.VMEM((2, _CH), jnp.int32),
            pltpu.VMEM((2, _CH), jnp.int32),
            pltpu.VMEM((2, _CH, _D1), jnp.float32),
            pltpu.VMEM((2, _CH, _D1), jnp.float32),
            pltpu.SemaphoreType.DMA,
            pltpu.SemaphoreType.DMA,
        ],
    )
    def k(a_hbm, bc_hbm, row_hbm, col_hbm, g_hbm,
          idxr, idxc, r1, r2, sg0, sg1):
        wid = lax.axis_index("s") * _NC + lax.axis_index("c")
        sems = (sg0, sg1)

        def fetch(t, b):
            off = (wid + t * _NW) * _CH
            pltpu.sync_copy(row_hbm.at[pl.ds(off, _CH)], idxr.at[b])
            pltpu.sync_copy(col_hbm.at[pl.ds(off, _CH)], idxc.at[b])
            pltpu.async_copy(a_hbm.at[idxr.at[b]], r1.at[b], sems[b])
            pltpu.async_copy(bc_hbm.at[idxc.at[b]], r2.at[b], sems[b])

        def drain_wb(t, b):
            off = (wid + t * _NW) * _CH
            pltpu.make_async_copy(a_hbm.at[idxr.at[b]], r1.at[b], sems[b]).wait()
            pltpu.make_async_copy(bc_hbm.at[idxc.at[b]], r2.at[b], sems[b]).wait()

            @pl.loop(0, _CH, unroll=8)
            def _acc(e, b=b):
                for j in range(_D2 // 16):
                    r2[b, e, pl.ds(j * 16, 16)] = (
                        r2[b, e, pl.ds(j * 16, 16)]
                        + r1[b, e, pl.ds(j * 16, 16)])

            pltpu.sync_copy(r2.at[b], g_hbm.at[pl.ds(off, _CH)])

        fetch(0, 0)

        @pl.loop(0, trips, step=2)
        def _outer(t0):
            for b in (0, 1):
                t = t0 + b

                @pl.when(wid + (t + 1) * _NW < nchunk)
                def _(t=t, b=b):
                    fetch(t + 1, 1 - b)

                @pl.when(wid + t * _NW < nchunk)
                def _(t=t, b=b):
                    drain_wb(t, b)

    return k(a_tab, bc_tab, row, col)


def _edge_mlp(g, edge_attr, we1c, we2, wn1b, be1, be2, bn1w):
    be = 2000

    def body(g_ref, ea_ref, we1c_ref, we2_ref, wn1b_ref,
             be1_ref, be2_ref, bn1w_ref, h2_ref, wc_s, bc_s):
        @pl.when(pl.program_id(0) == 0)
        def _():
            wc_s[...] = jnp.dot(we2_ref[...], wn1b_ref[...],
                                preferred_element_type=jnp.float32)
            bc_s[...] = jnp.dot(be2_ref[...], wn1b_ref[...],
                                preferred_element_type=jnp.float32) + bn1w_ref[...]

        h1 = jnp.maximum(
            g_ref[:, :_D2]
            + jnp.dot(ea_ref[...], we1c_ref[...],
                      preferred_element_type=jnp.float32)
            + be1_ref[...], 0.0)
        h2 = jnp.maximum(
            g_ref[:, _D2:] + jnp.dot(h1, wc_s[...],
                                     preferred_element_type=jnp.float32)
            + bc_s[...], 0.0)
        lane = lax.broadcasted_iota(jnp.int32, (be, _D1), 1)
        h2_ref[...] = jnp.concatenate(
            [h2, (lane[:, _D2:] == _D2).astype(jnp.float32)], axis=1)

    return pl.pallas_call(
        body,
        grid=(g.shape[0] // be,),
        in_specs=[pl.BlockSpec((be, _D1), lambda i: (i, 0)),
                  pl.BlockSpec((be, _D1), lambda i: (i, 0)),
                  pl.BlockSpec((_D1, _D2), lambda i: (0, 0)),
                  pl.BlockSpec((_D2, _D3), lambda i: (0, 0)),
                  pl.BlockSpec((_D3, _D2), lambda i: (0, 0)),
                  pl.BlockSpec((1, _D2), lambda i: (0, 0)),
                  pl.BlockSpec((1, _D3), lambda i: (0, 0)),
                  pl.BlockSpec((1, _D2), lambda i: (0, 0))],
        out_specs=pl.BlockSpec((be, _D1), lambda i: (i, 0)),
        out_shape=jax.ShapeDtypeStruct((g.shape[0], _D1), jnp.float32),
        scratch_shapes=[pltpu.VMEM((_D2, _D2), jnp.float32),
                        pltpu.VMEM((1, _D2), jnp.float32)],
    )(g, edge_attr, we1c, we2, wn1b, be1, be2, bn1w)


def _sc_scatter(h2s, rows, zs):
    rpt = _NP // _NS

    @functools.partial(
        pl.kernel,
        out_type=jax.ShapeDtypeStruct((_NC, _NP, _D1), jnp.float32),
        mesh=_sc_mesh(),
        scratch_types=[
            pltpu.VMEM((2, _CH), jnp.int32),
            pltpu.VMEM((2, _CH, _D1), jnp.float32),
            pltpu.VMEM_SHARED((_NP, _D1), jnp.float32),
            pltpu.SemaphoreType.DMA,
            pltpu.SemaphoreType.DMA,
        ],
    )
    def k(*refs):
        h2_hbms = refs[0:_NSL]
        row_hbms = refs[_NSL:2 * _NSL]
        zs_hbm = refs[2 * _NSL]
        sp_hbm = refs[2 * _NSL + 1]
        idx, hv, s_sh, sh0, sh1 = refs[2 * _NSL + 2:]
        c = lax.axis_index("c")
        s = lax.axis_index("s")
        wid = s * _NC + c
        sems = (sh0, sh1)
        pltpu.sync_copy(zs_hbm.at[pl.ds(s * rpt, rpt)], s_sh.at[pl.ds(s * rpt, rpt)])
        plsc.subcore_barrier()

        def fetch(sub, t, b):
            off = (wid + t * _NW) * _CH
            pltpu.sync_copy(row_hbms[sub].at[pl.ds(off, _CH)], idx.at[b])
            pltpu.async_copy(h2_hbms[sub].at[pl.ds(off, _CH)], hv.at[b], sems[b])

        def drain_add(sub, b):
            pltpu.make_async_copy(h2_hbms[sub].at[pl.ds(0, _CH)], hv.at[b],
                                  sems[b]).wait()
            pltpu.sync_copy(hv.at[b], s_sh.at[idx.at[b]], add=True)

        for sub in range(_NSL):
            nchunk = _SLICES[sub] // _CH
            trips = -(-nchunk // _NW)
            fetch(sub, 0, 0)

            @pl.loop(0, trips, step=2)
            def _outer(t0, sub=sub, nchunk=nchunk):
                for b in (0, 1):
                    t = t0 + b

                    @pl.when(wid + (t + 1) * _NW < nchunk)
                    def _(t=t, b=b, sub=sub, nchunk=nchunk):
                        fetch(sub, t + 1, 1 - b)

                    @pl.when(wid + t * _NW < nchunk)
                    def _(t=t, b=b, sub=sub, nchunk=nchunk):
                        drain_add(sub, b)

        plsc.subcore_barrier()
        pltpu.sync_copy(s_sh.at[pl.ds(s * rpt, rpt)],
                        sp_hbm.at[c, pl.ds(s * rpt, rpt)])

    return k(*h2s, *rows, zs)


def _final(sparts, batch3, u, wn2, bn2w, wg1u, wg1g, bg1, wg2, bg2,
           wfc1, bfc1, gamma, beta, wfc2p, bfc2p):
    bf = 1000
    nblk = _N // bf

    def body(sp_ref, b_ref, u_ref, wn2_ref, bn2w_ref, wg1u_ref,
             wg1g_ref, bg1_ref, wg2_ref, bg2_ref, wfc1_ref, bfc1_ref,
             gam_ref, bet_ref, wfc2_ref, bfc2_ref, out_ref,
             accr, accm, acca):
        i = pl.program_id(0)

        @pl.when(i == 0)
        def _():
            accr[...] = jnp.zeros_like(accr)
            accm[...] = jnp.zeros_like(accm)
            acca[...] = jnp.zeros_like(acca)

        st = sp_ref[0] + sp_ref[1]
        s = st[:, :_D2]
        cnt = st[:, _D2:_D2 + 1]
        mh = s / jnp.maximum(cnt, 1.0)
        nz = (cnt > 0.0).astype(jnp.float32)
        x2 = jnp.dot(mh, wn2_ref[...], preferred_element_type=jnp.float32) \
            + nz * bn2w_ref[...]
        r = jnp.maximum(x2, 0.0)
        bvals = b_ref[0, 0, :]
        onehot = (bvals[None, :]
                  == lax.broadcasted_iota(jnp.int32, (_B, bf), 0)
                  ).astype(jnp.float32)
        accr[...] += jnp.dot(onehot, r, preferred_element_type=jnp.float32)
        accm[...] += jnp.dot(onehot, mh, preferred_element_type=jnp.float32)
        aux = jnp.concatenate(
            [jnp.ones((bf, _D2), jnp.float32),
             jnp.broadcast_to(nz, (bf, _D2))], axis=1)
        acca[...] += jnp.dot(onehot, aux, preferred_element_type=jnp.float32)

        @pl.when(i == nblk - 1)
        def _():
            nb = acca[:, 0:1]
            nzc = acca[:, _D2:_D2 + 1]
            gp = (jnp.dot(accm[...], wn2_ref[...],
                          preferred_element_type=jnp.float32)
                  + nzc * bn2w_ref[...]) / jnp.maximum(nb, 1.0)
            g1h = jnp.maximum(
                u_ref[...] * wg1u_ref[...]
                + jnp.dot(gp, wg1g_ref[...], preferred_element_type=jnp.float32)
                + bg1_ref[...], 0.0)
            u2 = jnp.dot(g1h, wg2_ref[...],
                         preferred_element_type=jnp.float32) + bg2_ref[...]
            pooled = (accr[...] + jnp.maximum(u2, 0.0)) / (nb + 1.0)
            h = jnp.dot(pooled, wfc1_ref[...],
                        preferred_element_type=jnp.float32) + bfc1_ref[...]
            h = h * (1.0 / jnp.sqrt(1.0 + 1e-5)) * gam_ref[...] + bet_ref[...]
            h = jnp.maximum(h, 0.0)
            logits = jnp.dot(h, wfc2_ref[...],
                             preferred_element_type=jnp.float32) + bfc2_ref[...]
            colmask = lax.broadcasted_iota(jnp.int32, (_B, 128), 1) < 6
            lm = jnp.where(colmask, logits, -1e30)
            mx = jnp.max(lm, axis=1, keepdims=True)
            lse = jnp.log(jnp.sum(jnp.exp(lm - mx), axis=1, keepdims=True)) + mx
            out_ref[...] = lm - lse

    return pl.pallas_call(
        body,
        grid=(nblk,),
        in_specs=[pl.BlockSpec((_NC, bf, _D1), lambda i: (0, i, 0)),
                  pl.BlockSpec((1, 1, bf), lambda i: (i, 0, 0)),
                  pl.BlockSpec((_B, 1), lambda i: (0, 0)),
                  pl.BlockSpec((_D2, _D3), lambda i: (0, 0)),
                  pl.BlockSpec((1, _D3), lambda i: (0, 0)),
                  pl.BlockSpec((1, _D2), lambda i: (0, 0)),
                  pl.BlockSpec((_D3, _D2), lambda i: (0, 0)),
                  pl.BlockSpec((1, _D2), lambda i: (0, 0)),
                  pl.BlockSpec((_D2, _D3), lambda i: (0, 0)),
                  pl.BlockSpec((1, _D3), lambda i: (0, 0)),
                  pl.BlockSpec((_D3, _D2), lambda i: (0, 0)),
                  pl.BlockSpec((1, _D2), lambda i: (0, 0)),
                  pl.BlockSpec((1, _D2), lambda i: (0, 0)),
                  pl.BlockSpec((1, _D2), lambda i: (0, 0)),
                  pl.BlockSpec((_D2, 128), lambda i: (0, 0)),
                  pl.BlockSpec((1, 128), lambda i: (0, 0))],
        out_specs=pl.BlockSpec((_B, 128), lambda i: (0, 0)),
        out_shape=jax.ShapeDtypeStruct((_B, 128), jnp.float32),
        scratch_shapes=[pltpu.VMEM((_B, _D3), jnp.float32),
                        pltpu.VMEM((_B, _D2), jnp.float32),
                        pltpu.VMEM((_B, 128), jnp.float32)],
    )(sparts, batch3, u, wn2, bn2w, wg1u, wg1g, bg1, wg2, bg2,
      wfc1, bfc1, gamma, beta, wfc2p, bfc2p)


def kernel(x, edge_index, edge_attr, u, batch, We1, be1, We2, be2, Wn1, bn1w,
           Wn2, bn2w, Wg1, bg1, Wg2, bg2, Wfc1, bfc1, gamma, beta, Wfc2, bfc2):
    row = edge_index[0]
    col = edge_index[1]
    wcat = jnp.concatenate([We1[:_D1], We1[_D1:2 * _D1], Wn1[:_D1]], axis=1)

    a_tab, bc_tab = _nodeproj(x, wcat)
    h2s, rows = [], []
    off = 0
    for c in range(_NSL):
        es = _SLICES[c]
        rc = lax.slice_in_dim(row, off, off + es)
        cc = lax.slice_in_dim(col, off, off + es)
        g = _sc_gather(a_tab, bc_tab, rc, cc)
        eac = lax.slice_in_dim(edge_attr, off, off + es)
        h2s.append(_edge_mlp(g, eac, We1[2 * _D1:], We2, Wn1[_D1:],
                             be1.reshape(1, _D2), be2.reshape(1, _D3),
                             bn1w.reshape(1, _D2)))
        rows.append(rc)
        off += es
    zs = jnp.zeros((_NP, _D1), jnp.float32)
    sparts = _sc_scatter(h2s, rows, zs)

    batch3 = batch.reshape(_N // 1000, 1, 1000)
    out = _final(sparts, batch3, u, Wn2, bn2w.reshape(1, _D3),
                 Wg1[0:1], Wg1[1:], bg1.reshape(1, _D2), Wg2,
                 bg2.reshape(1, _D3), Wfc1, bfc1.reshape(1, _D2),
                 gamma.reshape(1, _D2), beta.reshape(1, _D2),
                 jnp.pad(Wfc2, ((0, 0), (0, 122))),
                 jnp.pad(bfc2, (0, 122)).reshape(1, 128))
    return out[:, :6]
